# baseline (speedup 1.0000x reference)
import jax
import jax.numpy as jnp
from jax import lax
from jax.experimental import pallas as pl
from jax.experimental.pallas import tpu as pltpu
from jax.experimental.pallas import tpu_sc as plsc

_B, _MQ, _QL, _V, _E, _H, _DV = 16, 8, 16, 5000, 512, 512, 1024
_S = _MQ * _QL
_N = _S * _B


def _sc_gather(table, idx):
    n, d = idx.shape[0], table.shape[1]
    nw = 32
    per = n // nw
    mesh = plsc.VectorSubcoreMesh(core_axis_name="c", subcore_axis_name="s")

    def body(table_hbm, idx_hbm, out_hbm, idx_v, rows_v, sem):
        wid = lax.axis_index("s") * 2 + lax.axis_index("c")
        base = wid * per
        pltpu.sync_copy(idx_hbm.at[pl.ds(base, per)], idx_v)
        pltpu.async_copy(table_hbm.at[idx_v], rows_v, sem).wait()
        pltpu.sync_copy(rows_v, out_hbm.at[pl.ds(base, per)])

    return pl.kernel(
        body,
        out_type=jax.ShapeDtypeStruct((n, d), table.dtype),
        mesh=mesh,
        scratch_types=[
            pltpu.VMEM((per,), jnp.int32),
            pltpu.VMEM((per, d), table.dtype),
            pltpu.SemaphoreType.DMA,
        ],
    )(table, idx)


def _mmT(x, w):
    return lax.dot_general(x, w, (((1,), (1,)), ((), ())),
                           preferred_element_type=jnp.float32)


def _vis_pre(vis, wiv, b2d):
    def body(v_ref, w_ref, b_ref, o_ref):
        o_ref[...] = _mmT(v_ref[...], w_ref[...]) + b_ref[...]

    return pl.pallas_call(
        body,
        out_shape=jax.ShapeDtypeStruct((_B, 4 * _H), jnp.float32),
    )(vis, wiv, b2d)


def _lstm_scan(xe3, wieT, whhT, visz, sel):
    def body(xe_ref, wie_ref, whh_ref, vz_ref, sel_ref, hs_ref, ch_ref, cc_ref):
        qi = pl.program_id(0)

        @pl.when(qi == 0)
        def _():
            ch_ref[...] = jnp.zeros_like(ch_ref)
            cc_ref[...] = jnp.zeros_like(cc_ref)

        xe = xe_ref[...].reshape(_QL * _B, _E)
        z0 = _mmT(xe, wie_ref[...])
        z0 = z0.reshape(_QL, _B, 4 * _H) + vz_ref[...][None]
        whh = whh_ref[...]
        h = ch_ref[...]
        c = cc_ref[...]
        carry_h = h
        carry_c = c
        for t in range(_QL):
            z = z0[t] + _mmT(h, whh)
            zi = z[:, 0 * _H:1 * _H]
            zf = z[:, 1 * _H:2 * _H]
            zg = z[:, 2 * _H:3 * _H]
            zo = z[:, 3 * _H:4 * _H]
            c = jax.nn.sigmoid(zf) * c + jax.nn.sigmoid(zi) * jnp.tanh(zg)
            h = jax.nn.sigmoid(zo) * jnp.tanh(c)
            hs_ref[:, t, :] = h
            s = sel_ref[0, :, t:t + 1]
            carry_h = s * h + (1.0 - s) * carry_h
            carry_c = s * c + (1.0 - s) * carry_c
        ch_ref[...] = carry_h
        cc_ref[...] = carry_c

    return pl.pallas_call(
        body,
        grid=(_MQ,),
        in_specs=[
            pl.BlockSpec((_QL, _B, _E), lambda i: (i, 0, 0)),
            pl.BlockSpec((4 * _H, _E), lambda i: (0, 0)),
            pl.BlockSpec((4 * _H, _H), lambda i: (0, 0)),
            pl.BlockSpec((_B, 4 * _H), lambda i: (0, 0)),
            pl.BlockSpec((1, _B, _QL), lambda i: (i, 0, 0)),
        ],
        out_specs=pl.BlockSpec((_B, _QL, _H), lambda i: (0, i, 0)),
        out_shape=jax.ShapeDtypeStruct((_B, _S, _H), jnp.float32),
        scratch_shapes=[
            pltpu.VMEM((_B, _H), jnp.float32),
            pltpu.VMEM((_B, _H), jnp.float32),
        ],
    )(xe3, wieT, whhT, visz, sel)


def _proj(hs2, outW, outb2, vmask):
    tr, tc = 256, 640
    grid = (_N // tr, (_V + tc - 1) // tc)

    def body(h_ref, w_ref, b_ref, m_ref, o_ref):
        o_ref[...] = (_mmT(h_ref[...], w_ref[...]) + b_ref[...]) * m_ref[...]

    return pl.pallas_call(
        body,
        grid=grid,
        in_specs=[
            pl.BlockSpec((tr, _H), lambda i, j: (i, 0)),
            pl.BlockSpec((tc, _H), lambda i, j: (j, 0)),
            pl.BlockSpec((1, tc), lambda i, j: (0, j)),
            pl.BlockSpec((tr, 1), lambda i, j: (i, 0)),
        ],
        out_specs=pl.BlockSpec((tr, tc), lambda i, j: (i, j)),
        out_shape=jax.ShapeDtypeStruct((_N, _V), jnp.float32),
    )(hs2, outW, outb2, vmask)


def kernel(source_questions, question_lengths, visual_features, unrolled_dialogue,
           cumulative_lengths, num_questions, object_categories, object_bboxes,
           emb, Wih, Whh, b, outW, outb, g_emb, g_Wih, g_Whh, g_b,
           cat_emb, W1, b1, W2, b2):
    toks = source_questions.transpose(1, 2, 0).reshape(_N).astype(jnp.int32)
    xe = _sc_gather(emb, toks)
    xe3 = xe.reshape(_S, _B, _E)

    wie = Wih[:, :_E]
    wiv = Wih[:, _E:]
    visz = _vis_pre(visual_features, wiv, b.reshape(1, 4 * _H))

    lens = question_lengths.astype(jnp.int32)
    nq = num_questions.astype(jnp.int32)
    running = jnp.arange(_MQ)[None, :] < nq[:, None]
    tix = jnp.arange(_QL)
    sel = (lens[:, :, None] - 1 == tix[None, None, :]) & running[:, :, None]
    sel = sel.transpose(1, 0, 2).astype(jnp.float32)
    valid = (tix[None, None, :] < lens[:, :, None]) & running[:, :, None]
    vmask = valid.reshape(_N, 1).astype(jnp.float32)

    hs = _lstm_scan(xe3, wie, Whh, visz, sel)
    hs2 = hs.reshape(_N, _H)
    return _proj(hs2, outW, outb.reshape(1, _V), vmask)


# speedup vs baseline: 1.4655x; 1.4655x over previous
import jax
import jax.numpy as jnp
from jax import lax
from jax.experimental import pallas as pl
from jax.experimental.pallas import tpu as pltpu
from jax.experimental.pallas import tpu_sc as plsc

_B, _MQ, _QL, _V, _E, _H, _DV = 16, 8, 16, 5000, 512, 512, 1024
_S = _MQ * _QL
_N = _S * _B


def _sc_gather(table, idx):
    n, d = idx.shape[0], table.shape[1]
    nw = 32
    per = n // nw
    mesh = plsc.VectorSubcoreMesh(core_axis_name="c", subcore_axis_name="s")

    def body(table_hbm, idx_hbm, out_hbm, idx_v, rows_v, sem):
        wid = lax.axis_index("s") * 2 + lax.axis_index("c")
        base = wid * per
        pltpu.sync_copy(idx_hbm.at[pl.ds(base, per)], idx_v)
        pltpu.async_copy(table_hbm.at[idx_v], rows_v, sem).wait()
        pltpu.sync_copy(rows_v, out_hbm.at[pl.ds(base, per)])

    return pl.kernel(
        body,
        out_type=jax.ShapeDtypeStruct((n, d), table.dtype),
        mesh=mesh,
        scratch_types=[
            pltpu.VMEM((per,), jnp.int32),
            pltpu.VMEM((per, d), table.dtype),
            pltpu.SemaphoreType.DMA,
        ],
    )(table, idx)


def _vis_pre(vis, wivT, b2d):
    def body(v_ref, w_ref, b_ref, o_ref):
        o_ref[...] = jnp.dot(v_ref[...], w_ref[...],
                             preferred_element_type=jnp.float32) + b_ref[...]

    return pl.pallas_call(
        body,
        out_shape=jax.ShapeDtypeStruct((_B, 4 * _H), jnp.float32),
    )(vis, wivT, b2d)


def _scan_proj(xe3, wieT, whhT, outWT, visz, sel, vmask3, outb2):
    def body(xe_ref, wie_ref, whh_ref, ow_ref, vz_ref, sel_ref, vm_ref,
             ob_ref, out_ref, ch_ref, cc_ref, hs_ref):
        qi = pl.program_id(0)

        @pl.when(qi == 0)
        def _():
            ch_ref[...] = jnp.zeros_like(ch_ref)
            cc_ref[...] = jnp.zeros_like(cc_ref)

        xe = xe_ref[...].reshape(_QL * _B, _E)
        z0 = jnp.dot(xe, wie_ref[...], preferred_element_type=jnp.float32)
        z0 = z0.reshape(_QL, _B, 4 * _H) + vz_ref[...][None]
        whh = whh_ref[...]
        h = ch_ref[...]
        c = cc_ref[...]
        carry_h = h
        carry_c = c
        for t in range(_QL):
            z = z0[t] + jnp.dot(h, whh, preferred_element_type=jnp.float32)
            zi = z[:, 0 * _H:1 * _H]
            zf = z[:, 1 * _H:2 * _H]
            zg = z[:, 2 * _H:3 * _H]
            zo = z[:, 3 * _H:4 * _H]
            c = jax.nn.sigmoid(zf) * c + jax.nn.sigmoid(zi) * jnp.tanh(zg)
            h = jax.nn.sigmoid(zo) * jnp.tanh(c)
            hs_ref[:, t, :] = h
            s = sel_ref[0, :, t:t + 1]
            carry_h = s * h + (1.0 - s) * carry_h
            carry_c = s * c + (1.0 - s) * carry_c
        ch_ref[...] = carry_h
        cc_ref[...] = carry_c

        hs2 = hs_ref[...].reshape(_B * _QL, _H)
        logits = jnp.dot(hs2, ow_ref[...], preferred_element_type=jnp.float32)
        logits = (logits + ob_ref[...]) * vm_ref[0]
        out_ref[...] = logits.reshape(_B, _QL, _V)

    return pl.pallas_call(
        body,
        grid=(_MQ,),
        in_specs=[
            pl.BlockSpec((_QL, _B, _E), lambda i: (i, 0, 0)),
            pl.BlockSpec((_E, 4 * _H), lambda i: (0, 0)),
            pl.BlockSpec((_H, 4 * _H), lambda i: (0, 0)),
            pl.BlockSpec((_H, _V), lambda i: (0, 0)),
            pl.BlockSpec((_B, 4 * _H), lambda i: (0, 0)),
            pl.BlockSpec((1, _B, _QL), lambda i: (i, 0, 0)),
            pl.BlockSpec((1, _B * _QL, 1), lambda i: (i, 0, 0)),
            pl.BlockSpec((1, _V), lambda i: (0, 0)),
        ],
        out_specs=pl.BlockSpec((_B, _QL, _V), lambda i: (0, i, 0)),
        out_shape=jax.ShapeDtypeStruct((_B, _S, _V), jnp.float32),
        scratch_shapes=[
            pltpu.VMEM((_B, _H), jnp.float32),
            pltpu.VMEM((_B, _H), jnp.float32),
            pltpu.VMEM((_B, _QL, _H), jnp.float32),
        ],
    )(xe3, wieT, whhT, outWT, visz, sel, vmask3, outb2)


def kernel(source_questions, question_lengths, visual_features, unrolled_dialogue,
           cumulative_lengths, num_questions, object_categories, object_bboxes,
           emb, Wih, Whh, b, outW, outb, g_emb, g_Wih, g_Whh, g_b,
           cat_emb, W1, b1, W2, b2):
    toks = source_questions.transpose(1, 2, 0).reshape(_N).astype(jnp.int32)
    xe = _sc_gather(emb, toks)
    xe3 = xe.reshape(_S, _B, _E)

    wieT = Wih[:, :_E].T
    wivT = Wih[:, _E:].T
    visz = _vis_pre(visual_features, wivT, b.reshape(1, 4 * _H))

    lens = question_lengths.astype(jnp.int32)
    nq = num_questions.astype(jnp.int32)
    running = jnp.arange(_MQ)[None, :] < nq[:, None]
    tix = jnp.arange(_QL)
    sel = (lens[:, :, None] - 1 == tix[None, None, :]) & running[:, :, None]
    sel = sel.transpose(1, 0, 2).astype(jnp.float32)
    valid = (tix[None, None, :] < lens[:, :, None]) & running[:, :, None]
    vmask3 = valid.transpose(1, 0, 2).reshape(_MQ, _B * _QL, 1)
    vmask3 = vmask3.astype(jnp.float32)

    out = _scan_proj(xe3, wieT, Whh.T, outW.T, visz, sel, vmask3,
                     outb.reshape(1, _V))
    return out.reshape(_N, _V)


# speedup vs baseline: 1.6756x; 1.1434x over previous
import jax
import jax.numpy as jnp
from jax import lax
from jax.experimental import pallas as pl
from jax.experimental.pallas import tpu as pltpu
from jax.experimental.pallas import tpu_sc as plsc

_B, _MQ, _QL, _V, _E, _H, _DV = 16, 8, 16, 5000, 512, 512, 1024
_S = _MQ * _QL
_N = _S * _B


def _sc_gather(table, idx):
    n, d = idx.shape[0], table.shape[1]
    nw = 32
    per = n // nw
    mesh = plsc.VectorSubcoreMesh(core_axis_name="c", subcore_axis_name="s")

    def body(table_hbm, idx_hbm, out_hbm, idx_v, rows_v, sem):
        wid = lax.axis_index("s") * 2 + lax.axis_index("c")
        base = wid * per
        pltpu.sync_copy(idx_hbm.at[pl.ds(base, per)], idx_v)
        pltpu.async_copy(table_hbm.at[idx_v], rows_v, sem).wait()
        pltpu.sync_copy(rows_v, out_hbm.at[pl.ds(base, per)])

    return pl.kernel(
        body,
        out_type=jax.ShapeDtypeStruct((n, d), table.dtype),
        mesh=mesh,
        scratch_types=[
            pltpu.VMEM((per,), jnp.int32),
            pltpu.VMEM((per, d), table.dtype),
            pltpu.SemaphoreType.DMA,
        ],
    )(table, idx)


def _scan_proj(xe3, Wih, Whh, outW, visT, b2d, sel, vmask3, outb2):
    def body(xe_ref, wih_ref, whh_ref, ow_ref, vt_ref, b_ref, sel_ref,
             vm_ref, ob_ref, out_ref,
             wieT_s, whhT_s, owT_s, vz_s, hs_ref, ch_ref, cc_ref):
        qi = pl.program_id(0)

        @pl.when(qi == 0)
        def _():
            wih = wih_ref[...]
            wieT_s[...] = wih[:, :_E].T
            whhT_s[...] = whh_ref[...].T
            owT_s[...] = ow_ref[...].T
            viszT = jnp.dot(wih[:, _E:], vt_ref[...],
                            preferred_element_type=jnp.float32)
            vz_s[...] = viszT.T + b_ref[...]
            ch_ref[...] = jnp.zeros_like(ch_ref)
            cc_ref[...] = jnp.zeros_like(cc_ref)

        xe = xe_ref[...].reshape(_QL * _B, _E)
        z0 = jnp.dot(xe, wieT_s[...], preferred_element_type=jnp.float32)
        z0 = z0.reshape(_QL, _B, 4 * _H) + vz_s[...][None]
        whhT = whhT_s[...]
        h = ch_ref[...]
        c = cc_ref[...]
        carry_h = h
        carry_c = c
        for t in range(_QL):
            z = z0[t] + jnp.dot(h, whhT, preferred_element_type=jnp.float32)
            zi = z[:, 0 * _H:1 * _H]
            zf = z[:, 1 * _H:2 * _H]
            zg = z[:, 2 * _H:3 * _H]
            zo = z[:, 3 * _H:4 * _H]
            c = jax.nn.sigmoid(zf) * c + jax.nn.sigmoid(zi) * jnp.tanh(zg)
            h = jax.nn.sigmoid(zo) * jnp.tanh(c)
            hs_ref[:, t, :] = h
            s = sel_ref[0, :, t:t + 1]
            carry_h = s * h + (1.0 - s) * carry_h
            carry_c = s * c + (1.0 - s) * carry_c
        ch_ref[...] = carry_h
        cc_ref[...] = carry_c

        hs2 = hs_ref[...].reshape(_B * _QL, _H).astype(jnp.bfloat16)
        logits = jnp.dot(hs2, owT_s[...], preferred_element_type=jnp.float32)
        logits = (logits + ob_ref[...]) * vm_ref[0]
        out_ref[...] = logits.reshape(_B, _QL, _V)

    return pl.pallas_call(
        body,
        grid=(_MQ,),
        in_specs=[
            pl.BlockSpec((_QL, _B, _E), lambda i: (i, 0, 0)),
            pl.BlockSpec((4 * _H, _E + _DV), lambda i: (0, 0)),
            pl.BlockSpec((4 * _H, _H), lambda i: (0, 0)),
            pl.BlockSpec((_V, _H), lambda i: (0, 0)),
            pl.BlockSpec((_DV, _B), lambda i: (0, 0)),
            pl.BlockSpec((1, 4 * _H), lambda i: (0, 0)),
            pl.BlockSpec((1, _B, _QL), lambda i: (i, 0, 0)),
            pl.BlockSpec((1, _B * _QL, 1), lambda i: (i, 0, 0)),
            pl.BlockSpec((1, _V), lambda i: (0, 0)),
        ],
        out_specs=pl.BlockSpec((_B, _QL, _V), lambda i: (0, i, 0)),
        out_shape=jax.ShapeDtypeStruct((_B, _S, _V), jnp.float32),
        scratch_shapes=[
            pltpu.VMEM((_E, 4 * _H), jnp.float32),
            pltpu.VMEM((_H, 4 * _H), jnp.float32),
            pltpu.VMEM((_H, _V), jnp.bfloat16),
            pltpu.VMEM((_B, 4 * _H), jnp.float32),
            pltpu.VMEM((_B, _QL, _H), jnp.float32),
            pltpu.VMEM((_B, _H), jnp.float32),
            pltpu.VMEM((_B, _H), jnp.float32),
        ],
    )(xe3, Wih, Whh, outW, visT, b2d, sel, vmask3, outb2)


def kernel(source_questions, question_lengths, visual_features, unrolled_dialogue,
           cumulative_lengths, num_questions, object_categories, object_bboxes,
           emb, Wih, Whh, b, outW, outb, g_emb, g_Wih, g_Whh, g_b,
           cat_emb, W1, b1, W2, b2):
    toks = source_questions.transpose(1, 2, 0).reshape(_N).astype(jnp.int32)
    xe = _sc_gather(emb, toks)
    xe3 = xe.reshape(_S, _B, _E)

    lens = question_lengths.astype(jnp.int32)
    nq = num_questions.astype(jnp.int32)
    running = jnp.arange(_MQ)[None, :] < nq[:, None]
    tix = jnp.arange(_QL)
    sel = (lens[:, :, None] - 1 == tix[None, None, :]) & running[:, :, None]
    sel = sel.transpose(1, 0, 2).astype(jnp.float32)
    valid = (tix[None, None, :] < lens[:, :, None]) & running[:, :, None]
    vmask3 = valid.transpose(1, 0, 2).reshape(_MQ, _B * _QL, 1)
    vmask3 = vmask3.astype(jnp.float32)

    out = _scan_proj(xe3, Wih, Whh, outW.astype(jnp.bfloat16),
                     visual_features.T,
                     b.reshape(1, 4 * _H), sel, vmask3,
                     outb.reshape(1, _V))
    return out.reshape(_N, _V)


# speedup vs baseline: 1.7435x; 1.0405x over previous
import jax
import jax.numpy as jnp
from jax import lax
from jax.experimental import pallas as pl
from jax.experimental.pallas import tpu as pltpu
from jax.experimental.pallas import tpu_sc as plsc

_B, _MQ, _QL, _V, _E, _H, _DV = 16, 8, 16, 5000, 512, 512, 1024
_S = _MQ * _QL
_N = _S * _B


def _sc_gather(table, idx):
    n, d = idx.shape[0], table.shape[1]
    nw = 32
    per = n // nw
    mesh = plsc.VectorSubcoreMesh(core_axis_name="c", subcore_axis_name="s")

    def body(table_hbm, idx_hbm, out_hbm, idx_v, rows_v, sem):
        wid = lax.axis_index("s") * 2 + lax.axis_index("c")
        base = wid * per
        pltpu.sync_copy(idx_hbm.at[pl.ds(base, per)], idx_v)
        pltpu.async_copy(table_hbm.at[idx_v], rows_v, sem).wait()
        pltpu.sync_copy(rows_v, out_hbm.at[pl.ds(base, per)])

    return pl.kernel(
        body,
        out_type=jax.ShapeDtypeStruct((n, d), table.dtype),
        mesh=mesh,
        scratch_types=[
            pltpu.VMEM((per,), jnp.int32),
            pltpu.VMEM((per, d), table.dtype),
            pltpu.SemaphoreType.DMA,
        ],
    )(table, idx)


def _scan_proj(xe3, Wih, Whh, outW, visT, b2d, sel, vmask3, outb2):
    def body(xe_ref, wih_ref, whh_ref, ow_ref, vt_ref, b_ref, sel_ref,
             vm_ref, ob_ref, out_ref,
             wieT_s, whhT_s, owT_s, vz_s, hs_ref, ch_ref, cc_ref):
        qi = pl.program_id(0)

        @pl.when(qi == 0)
        def _():
            wih = wih_ref[...]
            wieT_s[...] = wih[:, :_E].T
            whhT_s[...] = whh_ref[...].T
            owT_s[...] = ow_ref[...].T.astype(jnp.bfloat16)
            viszT = jnp.dot(wih[:, _E:], vt_ref[...],
                            preferred_element_type=jnp.float32)
            vz_s[...] = viszT.T + b_ref[...]
            ch_ref[...] = jnp.zeros_like(ch_ref)
            cc_ref[...] = jnp.zeros_like(cc_ref)

        xe = xe_ref[...].reshape(_QL * _B, _E)
        z0 = jnp.dot(xe, wieT_s[...], preferred_element_type=jnp.float32)
        z0 = z0.reshape(_QL, _B, 4 * _H) + vz_s[...][None]
        whhT = whhT_s[...]
        h = ch_ref[...]
        c = cc_ref[...]
        carry_h = h
        carry_c = c
        for t in range(_QL):
            z = z0[t] + jnp.dot(h, whhT, preferred_element_type=jnp.float32)
            zi = z[:, 0 * _H:1 * _H]
            zf = z[:, 1 * _H:2 * _H]
            zg = z[:, 2 * _H:3 * _H]
            zo = z[:, 3 * _H:4 * _H]
            c = jax.nn.sigmoid(zf) * c + jax.nn.sigmoid(zi) * jnp.tanh(zg)
            h = jax.nn.sigmoid(zo) * jnp.tanh(c)
            hs_ref[:, t, :] = h
            s = sel_ref[0, :, t:t + 1]
            carry_h = s * h + (1.0 - s) * carry_h
            carry_c = s * c + (1.0 - s) * carry_c
        ch_ref[...] = carry_h
        cc_ref[...] = carry_c

        hs2 = hs_ref[...].reshape(_B * _QL, _H).astype(jnp.bfloat16)
        logits = jnp.dot(hs2, owT_s[...], preferred_element_type=jnp.float32)
        logits = (logits + ob_ref[...]) * vm_ref[0]
        out_ref[...] = logits.reshape(_B, _QL, _V)

    return pl.pallas_call(
        body,
        grid=(_MQ,),
        in_specs=[
            pl.BlockSpec((_QL, _B, _E), lambda i: (i, 0, 0)),
            pl.BlockSpec((4 * _H, _E + _DV), lambda i: (0, 0)),
            pl.BlockSpec((4 * _H, _H), lambda i: (0, 0)),
            pl.BlockSpec((_V, _H), lambda i: (0, 0)),
            pl.BlockSpec((_DV, _B), lambda i: (0, 0)),
            pl.BlockSpec((1, 4 * _H), lambda i: (0, 0)),
            pl.BlockSpec((1, _B, _QL), lambda i: (i, 0, 0)),
            pl.BlockSpec((1, _B * _QL, 1), lambda i: (i, 0, 0)),
            pl.BlockSpec((1, _V), lambda i: (0, 0)),
        ],
        out_specs=pl.BlockSpec((_B, _QL, _V), lambda i: (0, i, 0)),
        out_shape=jax.ShapeDtypeStruct((_B, _S, _V), jnp.float32),
        scratch_shapes=[
            pltpu.VMEM((_E, 4 * _H), jnp.float32),
            pltpu.VMEM((_H, 4 * _H), jnp.float32),
            pltpu.VMEM((_H, _V), jnp.bfloat16),
            pltpu.VMEM((_B, 4 * _H), jnp.float32),
            pltpu.VMEM((_B, _QL, _H), jnp.float32),
            pltpu.VMEM((_B, _H), jnp.float32),
            pltpu.VMEM((_B, _H), jnp.float32),
        ],
    )(xe3, Wih, Whh, outW, visT, b2d, sel, vmask3, outb2)


def kernel(source_questions, question_lengths, visual_features, unrolled_dialogue,
           cumulative_lengths, num_questions, object_categories, object_bboxes,
           emb, Wih, Whh, b, outW, outb, g_emb, g_Wih, g_Whh, g_b,
           cat_emb, W1, b1, W2, b2):
    toks = source_questions.transpose(1, 2, 0).reshape(_N).astype(jnp.int32)
    xe = _sc_gather(emb, toks)
    xe3 = xe.reshape(_S, _B, _E)

    lens = question_lengths.astype(jnp.int32)
    nq = num_questions.astype(jnp.int32)
    running = jnp.arange(_MQ)[None, :] < nq[:, None]
    tix = jnp.arange(_QL)
    sel = (lens[:, :, None] - 1 == tix[None, None, :]) & running[:, :, None]
    sel = sel.transpose(1, 0, 2).astype(jnp.float32)
    valid = (tix[None, None, :] < lens[:, :, None]) & running[:, :, None]
    vmask3 = valid.transpose(1, 0, 2).reshape(_MQ, _B * _QL, 1)
    vmask3 = vmask3.astype(jnp.float32)

    out = _scan_proj(xe3, Wih, Whh, outW, visual_features.T,
                     b.reshape(1, 4 * _H), sel, vmask3,
                     outb.reshape(1, _V))
    return out.reshape(_N, _V)
